# baseline (speedup 1.0000x reference)
import functools

import jax
import jax.numpy as jnp
from jax import lax
from jax.experimental import pallas as pl
from jax.experimental.pallas import tpu as pltpu
from jax.experimental.pallas import tpu_sc as plsc

_B = 128
_N = 4096
_NC, _NS, _L = 2, 16, 16
_NW = _NC * _NS
_RPW = _B // _NW
_NCH = _N // _L
_NBINS = 4096
_CAND = _N + 2 * _L
_IMIN = -(2 ** 31)
_IMAX = 2 ** 31 - 1

_mesh = plsc.VectorSubcoreMesh(core_axis_name="c", subcore_axis_name="s",
                               num_cores=_NC, num_subcores=_NS)


@functools.partial(
    pl.kernel,
    out_type=jax.ShapeDtypeStruct((_B * 2,), jnp.int32),
    mesh=_mesh,
    compiler_params=pltpu.CompilerParams(needs_layout_passes=False),
    scratch_types=[
        pltpu.VMEM((_N,), jnp.int32),
        pltpu.VMEM((_NBINS,), jnp.int32),
        pltpu.VMEM((_CAND,), jnp.int32),
        pltpu.VMEM((_CAND,), jnp.int32),
        pltpu.VMEM((_N,), jnp.float32),
        pltpu.VMEM((_N,), jnp.int32),
        pltpu.VMEM((_B,), jnp.int32),
        pltpu.VMEM((_L,), jnp.int32),
    ],
)
def _sc_select(gumbel_hbm, absorbed_hbm, k_hbm, out_hbm,
               keyb, hist, candk, candi, grow, arow, kvec, outv):
    wid = lax.axis_index("s") * _NC + lax.axis_index("c")
    pltpu.sync_copy(k_hbm, kvec)
    lanes = lax.iota(jnp.int32, _L)
    ones = jnp.ones((_L,), jnp.int32)

    def row_body(j, ov):
        row = wid * _RPW + j
        pltpu.sync_copy(gumbel_hbm.at[row], grow)
        pltpu.sync_copy(absorbed_hbm.at[row], arow)

        kch = kvec[pl.ds((row // _L) * _L, _L)]
        k_in = jnp.sum(jnp.where(lanes == row % _L, kch, 0))

        def zbody(c, _):
            hist[pl.ds(c * _L, _L)] = jnp.zeros((_L,), jnp.int32)
            return 0
        lax.fori_loop(0, _NBINS // _L, zbody, 0)

        def s1body(c, carry):
            nabs, vmax = carry
            g = grow[pl.ds(c * _L, _L)]
            ab = arow[pl.ds(c * _L, _L)] != 0
            gi = plsc.bitcast(g, jnp.int32)
            ordk = jnp.where(gi < 0, gi ^ jnp.int32(0x7FFFFFFF), gi)
            ordk = jnp.where(g == 0.0, 0, ordk)
            key = jnp.where(ab, ordk, jnp.int32(_IMIN))
            keyb[pl.ds(c * _L, _L)] = key
            plsc.addupdate_scatter(hist, [(key >> 20) + 2048], ones)
            return (nabs + jnp.sum(ab.astype(jnp.int32)),
                    jnp.maximum(vmax, jnp.max(key)))
        nabs, kmax = lax.fori_loop(0, _NCH, s1body,
                                   (jnp.int32(0), jnp.int32(_IMIN)))
        k = jnp.minimum(nabs, k_in)

        c0 = ((kmax >> 20) + 2048) // _L

        def s2cond(st):
            c, cum = st[0], st[1]
            return (cum < k) & (c >= 0)

        def s2body(st):
            c, cum, b_star, above = st
            h = hist[pl.ds(c * _L, _L)]
            rev = lax.rev(h, (0,))
            cs = plsc.cumsum(rev) + cum
            hit = cs >= k
            any_hit = jnp.sum(hit.astype(jnp.int32)) > 0
            lane0 = jnp.max(plsc.all_reduce_ffs(hit))
            cs_at = jnp.sum(jnp.where(lanes == lane0, cs, 0))
            rv_at = jnp.sum(jnp.where(lanes == lane0, rev, 0))
            b_star = jnp.where(any_hit, c * _L + (_L - 1) - lane0, b_star)
            above = jnp.where(any_hit, cs_at - rv_at, above)
            return (c - 1, cum + jnp.sum(h), b_star, above)

        _, _, b_star, above = lax.while_loop(
            s2cond, s2body,
            (c0, jnp.int32(0), jnp.int32(0), jnp.int32(0)))
        need = k - above

        def s3body(c, off):
            key = keyb[pl.ds(c * _L, _L)]
            m = ((key >> 20) + 2048) == b_star
            mi = m.astype(jnp.int32)
            pos = off + plsc.cumsum(mi) - 1
            plsc.store_scatter(candk, [pos], key, mask=m)
            plsc.store_scatter(candi, [pos], lanes + c * _L, mask=m)
            return off + jnp.sum(mi)
        ncand = lax.fori_loop(0, _NCH, s3body, jnp.int32(0))
        plsc.store_scatter(candk, [ncand + lanes],
                           jnp.full((_L,), _IMIN, jnp.int32))
        nch4 = (ncand + _L - 1) // _L

        base = (b_star - 2048) << 20

        def count_ge(x):
            def cbody(c, acc):
                kk = candk[pl.ds(c * _L, _L)]
                return acc + jnp.sum((kk >= x).astype(jnp.int32))
            return lax.fori_loop(0, nch4, cbody, jnp.int32(0))

        def vdesc(i, prefix):
            bit = jnp.int32(1) << (19 - i)
            cnt = count_ge(base | prefix | bit)
            return jnp.where(cnt >= need, prefix | bit, prefix)
        prefix = lax.fori_loop(0, 20, vdesc, jnp.int32(0))
        t = base | prefix

        cnt_gt = count_ge(t + 1)
        ties_needed = need - cnt_gt

        def count_eq_le(ci):
            def cbody(c, acc):
                kk = candk[pl.ds(c * _L, _L)]
                ii = candi[pl.ds(c * _L, _L)]
                m = (kk == t) & (ii <= ci)
                return acc + jnp.sum(m.astype(jnp.int32))
            return lax.fori_loop(0, nch4, cbody, jnp.int32(0))

        def idesc(i, cut):
            bit = jnp.int32(1) << (11 - i)
            cte = count_eq_le(cut + bit - 1)
            return jnp.where(cte >= ties_needed, cut, cut + bit)
        cut = lax.fori_loop(0, 12, idesc, jnp.int32(0))

        tf = jnp.where(k > 0, t, jnp.int32(_IMAX))
        cf = jnp.where(k > 0, cut, jnp.int32(-1))
        ov = jnp.where(lanes == 2 * j, tf, ov)
        ov = jnp.where(lanes == 2 * j + 1, cf, ov)
        return ov

    ov = lax.fori_loop(0, _RPW, row_body, jnp.zeros((_L,), jnp.int32))
    outv[...] = ov
    pltpu.sync_copy(outv.at[pl.ds(0, 8)], out_hbm.at[pl.ds(wid * 8, 8)])


_ROWS = 16
_GRID = _B // _ROWS


def _tc_body(scores_ref, gumbel_ref, absorbed_ref, t_ref, c_ref, out_ref):
    scores = scores_ref[...]
    g = gumbel_ref[...]
    ab = absorbed_ref[...] != 0
    t = t_ref[...]
    cut = c_ref[...]

    gi = lax.bitcast_convert_type(g, jnp.int32)
    ordk = jnp.where(gi < 0, gi ^ jnp.int32(0x7FFFFFFF), gi)
    ordk = jnp.where(g == 0.0, jnp.int32(0), ordk)
    key = jnp.where(ab, ordk, jnp.int32(_IMIN))
    idx = lax.broadcasted_iota(jnp.int32, (_ROWS, _N), 1)
    sel = (key > t) | ((key == t) & (idx <= cut))

    m = jnp.max(scores, axis=1, keepdims=True)
    e = jnp.exp(scores - m)
    s = jnp.sum(e, axis=1, keepdims=True)
    out_ref[...] = jnp.where(sel, e / s, 0.0)


def kernel(scores, gumbel_noise, absorbed_mask, k_per_row):
    kk = k_per_row.astype(jnp.int32)
    sel = _sc_select(gumbel_noise, absorbed_mask, kk).reshape(_B, 2)
    t2 = sel[:, 0:1]
    c2 = sel[:, 1:2]
    return pl.pallas_call(
        _tc_body,
        grid=(_GRID,),
        in_specs=[
            pl.BlockSpec((_ROWS, _N), lambda i: (i, 0)),
            pl.BlockSpec((_ROWS, _N), lambda i: (i, 0)),
            pl.BlockSpec((_ROWS, _N), lambda i: (i, 0)),
            pl.BlockSpec((_ROWS, 1), lambda i: (i, 0)),
            pl.BlockSpec((_ROWS, 1), lambda i: (i, 0)),
        ],
        out_specs=pl.BlockSpec((_ROWS, _N), lambda i: (i, 0)),
        out_shape=jax.ShapeDtypeStruct((_B, _N), jnp.float32),
    )(scores, gumbel_noise, absorbed_mask, t2, c2)


# speedup vs baseline: 1.0172x; 1.0172x over previous
import functools

import jax
import jax.numpy as jnp
from jax import lax
from jax.experimental import pallas as pl
from jax.experimental.pallas import tpu as pltpu
from jax.experimental.pallas import tpu_sc as plsc

_B = 128
_N = 4096
_NC, _NS, _L = 2, 16, 16
_NW = _NC * _NS
_RPW = _B // _NW
_NCH = _N // _L
_NBINS = 4096
_CAND = _N + 2 * _L
_IMIN = -(2 ** 31)
_IMAX = 2 ** 31 - 1

_mesh = plsc.VectorSubcoreMesh(core_axis_name="c", subcore_axis_name="s",
                               num_cores=_NC, num_subcores=_NS)


@functools.partial(
    pl.kernel,
    out_type=jax.ShapeDtypeStruct((_B * 2,), jnp.int32),
    mesh=_mesh,
    compiler_params=pltpu.CompilerParams(needs_layout_passes=False),
    scratch_types=[
        pltpu.VMEM((_N,), jnp.int32),
        pltpu.VMEM((_NBINS,), jnp.int32),
        pltpu.VMEM((_CAND,), jnp.int32),
        pltpu.VMEM((_CAND,), jnp.int32),
        pltpu.VMEM((_N,), jnp.float32),
        pltpu.VMEM((_N,), jnp.int32),
        pltpu.VMEM((_B,), jnp.int32),
        pltpu.VMEM((_L,), jnp.int32),
    ],
)
def _sc_select(gumbel_hbm, absorbed_hbm, k_hbm, out_hbm,
               keyb, hist, candk, candi, grow, arow, kvec, outv):
    wid = lax.axis_index("s") * _NC + lax.axis_index("c")
    pltpu.sync_copy(k_hbm, kvec)
    lanes = lax.iota(jnp.int32, _L)
    ones = jnp.ones((_L,), jnp.int32)

    def zbody(c, _):
        hist[pl.ds(c * _L, _L)] = jnp.zeros((_L,), jnp.int32)
        return 0
    lax.fori_loop(0, _NBINS // _L, zbody, 0)

    def row_body(j, ov):
        row = wid * _RPW + j
        pltpu.sync_copy(gumbel_hbm.at[row], grow)
        pltpu.sync_copy(absorbed_hbm.at[row], arow)

        kch = kvec[pl.ds((row // _L) * _L, _L)]
        k_in = jnp.sum(jnp.where(lanes == row % _L, kch, 0))

        def s1body(cc, carry):
            nabs_v, vmax_v = carry
            for u in range(4):
                c = cc * 4 + u
                g = grow[pl.ds(c * _L, _L)]
                ab = arow[pl.ds(c * _L, _L)] != 0
                gi = plsc.bitcast(g, jnp.int32)
                ordk = jnp.where(gi < 0, gi ^ jnp.int32(0x7FFFFFFF), gi)
                ordk = jnp.where(g == 0.0, 0, ordk)
                key = jnp.where(ab, ordk, jnp.int32(_IMIN))
                keyb[pl.ds(c * _L, _L)] = key
                plsc.addupdate_scatter(hist, [(key >> 20) + 2048], ones)
                nabs_v = nabs_v + plsc.all_reduce_population_count(ab)
                vmax_v = jnp.maximum(vmax_v, key)
            return (nabs_v, vmax_v)
        nabs_v, vmax_v = lax.fori_loop(
            0, _NCH // 4, s1body,
            (jnp.zeros((_L,), jnp.int32), jnp.full((_L,), _IMIN, jnp.int32)))
        nabs = jnp.max(nabs_v)
        kmax = jnp.max(vmax_v)
        k = jnp.minimum(nabs, k_in)

        c0 = ((kmax >> 20) + 2048) // _L

        def s2cond(st):
            c, cum = st[0], st[1]
            return (cum < k) & (c >= 0)

        def s2body(st):
            c, cum, b_star, above = st
            h = hist[pl.ds(c * _L, _L)]
            rev = lax.rev(h, (0,))
            cs = plsc.cumsum(rev) + cum
            hit = cs >= k
            any_hit = jnp.sum(hit.astype(jnp.int32)) > 0
            lane0 = jnp.max(plsc.all_reduce_ffs(hit))
            cs_at = jnp.sum(jnp.where(lanes == lane0, cs, 0))
            rv_at = jnp.sum(jnp.where(lanes == lane0, rev, 0))
            b_star = jnp.where(any_hit, c * _L + (_L - 1) - lane0, b_star)
            above = jnp.where(any_hit, cs_at - rv_at, above)
            return (c - 1, cum + jnp.sum(h), b_star, above)

        _, _, b_star, above = lax.while_loop(
            s2cond, s2body,
            (c0, jnp.int32(0), jnp.int32(0), jnp.int32(0)))
        need = k - above

        def s3body(cc, off_v):
            for u in range(4):
                c = cc * 4 + u
                key = keyb[pl.ds(c * _L, _L)]
                b = (key >> 20) + 2048
                plsc.addupdate_scatter(hist, [b], -ones)
                m = b == b_star
                pos = off_v + plsc.cumsum(m.astype(jnp.int32)) - 1
                plsc.store_scatter(candk, [pos], key, mask=m)
                plsc.store_scatter(candi, [pos], lanes + c * _L, mask=m)
                off_v = off_v + plsc.all_reduce_population_count(m)
            return off_v
        ncand = jnp.max(lax.fori_loop(0, _NCH // 4, s3body,
                                      jnp.zeros((_L,), jnp.int32)))
        plsc.store_scatter(candk, [ncand + lanes],
                           jnp.full((_L,), _IMIN, jnp.int32))
        nch4 = (ncand + _L - 1) // _L

        base = (b_star - 2048) << 20

        def count_ge(x):
            def cbody(c, acc):
                kk = candk[pl.ds(c * _L, _L)]
                return acc + plsc.all_reduce_population_count(kk >= x)
            return jnp.max(lax.fori_loop(0, nch4, cbody,
                                         jnp.zeros((_L,), jnp.int32)))

        def vdesc(i, prefix):
            bit = jnp.int32(1) << (19 - i)
            cnt = count_ge(base | prefix | bit)
            return jnp.where(cnt >= need, prefix | bit, prefix)
        prefix = lax.fori_loop(0, 20, vdesc, jnp.int32(0))
        t = base | prefix

        cnt_gt = count_ge(t + 1)
        ties_needed = need - cnt_gt

        def count_eq_le(ci):
            def cbody(c, acc):
                kk = candk[pl.ds(c * _L, _L)]
                ii = candi[pl.ds(c * _L, _L)]
                m = (kk == t) & (ii <= ci)
                return acc + plsc.all_reduce_population_count(m)
            return jnp.max(lax.fori_loop(0, nch4, cbody,
                                         jnp.zeros((_L,), jnp.int32)))

        def idesc(i, cut):
            bit = jnp.int32(1) << (11 - i)
            cte = count_eq_le(cut + bit - 1)
            return jnp.where(cte >= ties_needed, cut, cut + bit)
        cut = lax.fori_loop(0, 12, idesc, jnp.int32(0))

        tf = jnp.where(k > 0, t, jnp.int32(_IMAX))
        cf = jnp.where(k > 0, cut, jnp.int32(-1))
        ov = jnp.where(lanes == 2 * j, tf, ov)
        ov = jnp.where(lanes == 2 * j + 1, cf, ov)
        return ov

    ov = lax.fori_loop(0, _RPW, row_body, jnp.zeros((_L,), jnp.int32))
    outv[...] = ov
    pltpu.sync_copy(outv.at[pl.ds(0, 8)], out_hbm.at[pl.ds(wid * 8, 8)])


_ROWS = 16
_GRID = _B // _ROWS


def _tc_body(scores_ref, gumbel_ref, absorbed_ref, t_ref, c_ref, out_ref):
    scores = scores_ref[...]
    g = gumbel_ref[...]
    ab = absorbed_ref[...] != 0
    t = t_ref[...]
    cut = c_ref[...]

    gi = lax.bitcast_convert_type(g, jnp.int32)
    ordk = jnp.where(gi < 0, gi ^ jnp.int32(0x7FFFFFFF), gi)
    ordk = jnp.where(g == 0.0, jnp.int32(0), ordk)
    key = jnp.where(ab, ordk, jnp.int32(_IMIN))
    idx = lax.broadcasted_iota(jnp.int32, (_ROWS, _N), 1)
    sel = (key > t) | ((key == t) & (idx <= cut))

    m = jnp.max(scores, axis=1, keepdims=True)
    e = jnp.exp(scores - m)
    s = jnp.sum(e, axis=1, keepdims=True)
    out_ref[...] = jnp.where(sel, e / s, 0.0)


def kernel(scores, gumbel_noise, absorbed_mask, k_per_row):
    kk = k_per_row.astype(jnp.int32)
    sel = _sc_select(gumbel_noise, absorbed_mask, kk).reshape(_B, 2)
    t2 = sel[:, 0:1]
    c2 = sel[:, 1:2]
    return pl.pallas_call(
        _tc_body,
        grid=(_GRID,),
        in_specs=[
            pl.BlockSpec((_ROWS, _N), lambda i: (i, 0)),
            pl.BlockSpec((_ROWS, _N), lambda i: (i, 0)),
            pl.BlockSpec((_ROWS, _N), lambda i: (i, 0)),
            pl.BlockSpec((_ROWS, 1), lambda i: (i, 0)),
            pl.BlockSpec((_ROWS, 1), lambda i: (i, 0)),
        ],
        out_specs=pl.BlockSpec((_ROWS, _N), lambda i: (i, 0)),
        out_shape=jax.ShapeDtypeStruct((_B, _N), jnp.float32),
    )(scores, gumbel_noise, absorbed_mask, t2, c2)


# speedup vs baseline: 1.0868x; 1.0685x over previous
import functools

import jax
import jax.numpy as jnp
from jax import lax
from jax.experimental import pallas as pl
from jax.experimental.pallas import tpu as pltpu
from jax.experimental.pallas import tpu_sc as plsc

_B = 128
_N = 4096
_NC, _NS, _L = 2, 16, 16
_NW = _NC * _NS
_RPW = _B // _NW
_NCH = _N // _L
_NBINS = 4096
_CAND = _N + 2 * _L
_IMIN = -(2 ** 31)
_IMAX = 2 ** 31 - 1

_mesh = plsc.VectorSubcoreMesh(core_axis_name="c", subcore_axis_name="s",
                               num_cores=_NC, num_subcores=_NS)


@functools.partial(
    pl.kernel,
    out_type=jax.ShapeDtypeStruct((_B * 2,), jnp.int32),
    mesh=_mesh,
    compiler_params=pltpu.CompilerParams(needs_layout_passes=False),
    scratch_types=[
        pltpu.VMEM((_N,), jnp.int32),
        pltpu.VMEM((_NBINS,), jnp.int32),
        pltpu.VMEM((_CAND,), jnp.int32),
        pltpu.VMEM((_CAND,), jnp.int32),
        pltpu.VMEM((_RPW, _N), jnp.float32),
        pltpu.VMEM((_RPW, _N), jnp.int32),
        pltpu.VMEM((_B,), jnp.int32),
        pltpu.VMEM((_L,), jnp.int32),
        pltpu.SemaphoreType.DMA,
        pltpu.SemaphoreType.DMA,
    ],
)
def _sc_select(gumbel_hbm, absorbed_hbm, k_hbm, out_hbm,
               keyb, hist, candk, candi, grows, arows, kvec, outv,
               sem_g, sem_a):
    wid = lax.axis_index("s") * _NC + lax.axis_index("c")
    row0 = wid * _RPW
    cp_g = pltpu.async_copy(gumbel_hbm.at[pl.ds(row0, _RPW)], grows, sem_g)
    cp_a = pltpu.async_copy(absorbed_hbm.at[pl.ds(row0, _RPW)], arows, sem_a)
    pltpu.sync_copy(k_hbm, kvec)
    lanes = lax.iota(jnp.int32, _L)
    ones = jnp.ones((_L,), jnp.int32)

    def zbody(c, _):
        hist[pl.ds(c * _L, _L)] = jnp.zeros((_L,), jnp.int32)
        return 0
    lax.fori_loop(0, _NBINS // _L, zbody, 0)
    cp_g.wait()
    cp_a.wait()

    def row_body(j, ov):
        row = row0 + j

        kch = kvec[pl.ds((row // _L) * _L, _L)]
        k_in = jnp.sum(jnp.where(lanes == row % _L, kch, 0))

        def s1body(cc, carry):
            nabs_v, vmax_v = carry
            for u in range(4):
                c = cc * 4 + u
                g = grows[j, pl.ds(c * _L, _L)]
                ab = arows[j, pl.ds(c * _L, _L)] != 0
                gi = plsc.bitcast(g, jnp.int32)
                ordk = jnp.where(gi < 0, gi ^ jnp.int32(0x7FFFFFFF), gi)
                ordk = jnp.where(g == 0.0, 0, ordk)
                key = jnp.where(ab, ordk, jnp.int32(_IMIN))
                keyb[pl.ds(c * _L, _L)] = key
                plsc.addupdate_scatter(hist, [(key >> 20) + 2048], ones)
                nabs_v = nabs_v + plsc.all_reduce_population_count(ab)
                vmax_v = jnp.maximum(vmax_v, key)
            return (nabs_v, vmax_v)
        nabs_v, vmax_v = lax.fori_loop(
            0, _NCH // 4, s1body,
            (jnp.zeros((_L,), jnp.int32), jnp.full((_L,), _IMIN, jnp.int32)))
        nabs = jnp.max(nabs_v)
        kmax = jnp.max(vmax_v)
        k = jnp.minimum(nabs, k_in)

        c0 = ((kmax >> 20) + 2048) // _L

        def s2cond(st):
            c, cum = st[0], st[1]
            return (cum < k) & (c >= 0)

        def s2body(st):
            c, cum, b_star, above = st
            h = hist[pl.ds(c * _L, _L)]
            rev = lax.rev(h, (0,))
            cs = plsc.cumsum(rev) + cum
            hit = cs >= k
            any_hit = jnp.sum(hit.astype(jnp.int32)) > 0
            lane0 = jnp.max(plsc.all_reduce_ffs(hit))
            cs_at = jnp.sum(jnp.where(lanes == lane0, cs, 0))
            rv_at = jnp.sum(jnp.where(lanes == lane0, rev, 0))
            b_star = jnp.where(any_hit, c * _L + (_L - 1) - lane0, b_star)
            above = jnp.where(any_hit, cs_at - rv_at, above)
            return (c - 1, cum + jnp.sum(h), b_star, above)

        _, _, b_star, above = lax.while_loop(
            s2cond, s2body,
            (c0, jnp.int32(0), jnp.int32(0), jnp.int32(0)))
        need = k - above

        def s3body(cc, off_v):
            for u in range(4):
                c = cc * 4 + u
                key = keyb[pl.ds(c * _L, _L)]
                b = (key >> 20) + 2048
                plsc.addupdate_scatter(hist, [b], -ones)
                m = b == b_star
                pos = off_v + plsc.cumsum(m.astype(jnp.int32)) - 1
                plsc.store_scatter(candk, [pos], key, mask=m)
                plsc.store_scatter(candi, [pos], lanes + c * _L, mask=m)
                off_v = off_v + plsc.all_reduce_population_count(m)
            return off_v
        ncand = jnp.max(lax.fori_loop(0, _NCH // 4, s3body,
                                      jnp.zeros((_L,), jnp.int32)))
        plsc.store_scatter(candk, [ncand + lanes],
                           jnp.full((_L,), _IMIN, jnp.int32))
        nch4 = (ncand + _L - 1) // _L

        base = (b_star - 2048) << 20

        def count_ge(x):
            def cbody(c, acc):
                kk = candk[pl.ds(c * _L, _L)]
                return acc + plsc.all_reduce_population_count(kk >= x)
            return jnp.max(lax.fori_loop(0, nch4, cbody,
                                         jnp.zeros((_L,), jnp.int32)))

        def vdesc(i, prefix):
            bit = jnp.int32(1) << (19 - i)
            cnt = count_ge(base | prefix | bit)
            return jnp.where(cnt >= need, prefix | bit, prefix)
        prefix = lax.fori_loop(0, 20, vdesc, jnp.int32(0))
        t = base | prefix

        cnt_gt = count_ge(t + 1)
        ties_needed = need - cnt_gt

        def count_eq_le(ci):
            def cbody(c, acc):
                kk = candk[pl.ds(c * _L, _L)]
                ii = candi[pl.ds(c * _L, _L)]
                m = (kk == t) & (ii <= ci)
                return acc + plsc.all_reduce_population_count(m)
            return jnp.max(lax.fori_loop(0, nch4, cbody,
                                         jnp.zeros((_L,), jnp.int32)))

        def idesc(i, cut):
            bit = jnp.int32(1) << (11 - i)
            cte = count_eq_le(cut + bit - 1)
            return jnp.where(cte >= ties_needed, cut, cut + bit)
        cut = lax.fori_loop(0, 12, idesc, jnp.int32(0))

        tf = jnp.where(k > 0, t, jnp.int32(_IMAX))
        cf = jnp.where(k > 0, cut, jnp.int32(-1))
        ov = jnp.where(lanes == 2 * j, tf, ov)
        ov = jnp.where(lanes == 2 * j + 1, cf, ov)
        return ov

    ov = lax.fori_loop(0, _RPW, row_body, jnp.zeros((_L,), jnp.int32))
    outv[...] = ov
    pltpu.sync_copy(outv.at[pl.ds(0, 8)], out_hbm.at[pl.ds(wid * 8, 8)])


_ROWS = 16
_GRID = _B // _ROWS


def _tc_body(scores_ref, gumbel_ref, absorbed_ref, t_ref, c_ref, out_ref):
    scores = scores_ref[...]
    g = gumbel_ref[...]
    ab = absorbed_ref[...] != 0
    t = t_ref[...]
    cut = c_ref[...]

    gi = lax.bitcast_convert_type(g, jnp.int32)
    ordk = jnp.where(gi < 0, gi ^ jnp.int32(0x7FFFFFFF), gi)
    ordk = jnp.where(g == 0.0, jnp.int32(0), ordk)
    key = jnp.where(ab, ordk, jnp.int32(_IMIN))
    idx = lax.broadcasted_iota(jnp.int32, (_ROWS, _N), 1)
    sel = (key > t) | ((key == t) & (idx <= cut))

    m = jnp.max(scores, axis=1, keepdims=True)
    e = jnp.exp(scores - m)
    s = jnp.sum(e, axis=1, keepdims=True)
    out_ref[...] = jnp.where(sel, e / s, 0.0)


def kernel(scores, gumbel_noise, absorbed_mask, k_per_row):
    kk = k_per_row.astype(jnp.int32)
    sel = _sc_select(gumbel_noise, absorbed_mask, kk).reshape(_B, 2)
    t2 = sel[:, 0:1]
    c2 = sel[:, 1:2]
    return pl.pallas_call(
        _tc_body,
        grid=(_GRID,),
        in_specs=[
            pl.BlockSpec((_ROWS, _N), lambda i: (i, 0)),
            pl.BlockSpec((_ROWS, _N), lambda i: (i, 0)),
            pl.BlockSpec((_ROWS, _N), lambda i: (i, 0)),
            pl.BlockSpec((_ROWS, 1), lambda i: (i, 0)),
            pl.BlockSpec((_ROWS, 1), lambda i: (i, 0)),
        ],
        out_specs=pl.BlockSpec((_ROWS, _N), lambda i: (i, 0)),
        out_shape=jax.ShapeDtypeStruct((_B, _N), jnp.float32),
    )(scores, gumbel_noise, absorbed_mask, t2, c2)


# speedup vs baseline: 1.4451x; 1.3297x over previous
import functools

import jax
import jax.numpy as jnp
from jax import lax
from jax.experimental import pallas as pl
from jax.experimental.pallas import tpu as pltpu
from jax.experimental.pallas import tpu_sc as plsc

_B = 128
_N = 4096
_NC, _NS, _L = 2, 16, 16
_NW = _NC * _NS
_RPW = _B // _NW
_NCH = _N // _L
_NBINS = 4096
_CAND = _N + 2 * _L
_IMIN = -(2 ** 31)
_IMAX = 2 ** 31 - 1

_mesh = plsc.VectorSubcoreMesh(core_axis_name="c", subcore_axis_name="s",
                               num_cores=_NC, num_subcores=_NS)


@functools.partial(
    pl.kernel,
    out_type=jax.ShapeDtypeStruct((_B * 2,), jnp.int32),
    mesh=_mesh,
    compiler_params=pltpu.CompilerParams(needs_layout_passes=False),
    scratch_types=[
        pltpu.VMEM((_N,), jnp.int32),
        pltpu.VMEM((_NBINS,), jnp.int32),
        pltpu.VMEM((_CAND,), jnp.int32),
        pltpu.VMEM((_CAND,), jnp.int32),
        pltpu.VMEM((_RPW, _N), jnp.float32),
        pltpu.VMEM((_RPW, _N), jnp.int32),
        pltpu.VMEM((_B,), jnp.int32),
        pltpu.VMEM((_L,), jnp.int32),
        pltpu.SemaphoreType.DMA,
        pltpu.SemaphoreType.DMA,
    ],
)
def _sc_select(gumbel_hbm, absorbed_hbm, k_hbm, out_hbm,
               keyb, hist, candk, candi, grows, arows, kvec, outv,
               sem_g, sem_a):
    wid = lax.axis_index("s") * _NC + lax.axis_index("c")
    row0 = wid * _RPW
    cp_g = pltpu.async_copy(gumbel_hbm.at[pl.ds(row0, _RPW)], grows, sem_g)
    cp_a = pltpu.async_copy(absorbed_hbm.at[pl.ds(row0, _RPW)], arows, sem_a)
    pltpu.sync_copy(k_hbm, kvec)
    lanes = lax.iota(jnp.int32, _L)
    ones = jnp.ones((_L,), jnp.int32)

    def zbody(c, _):
        hist[pl.ds(c * _L, _L)] = jnp.zeros((_L,), jnp.int32)
        return 0
    lax.fori_loop(0, _NBINS // _L, zbody, 0)
    cp_g.wait()
    cp_a.wait()

    def row_body(j, ov):
        row = row0 + j

        kch = kvec[pl.ds((row // _L) * _L, _L)]
        k_in = jnp.sum(jnp.where(lanes == row % _L, kch, 0))

        def s1body(cc, carry):
            nabs_v, vmax_v = carry
            for u in range(4):
                c = cc * 4 + u
                g = grows[j, pl.ds(c * _L, _L)]
                ab = arows[j, pl.ds(c * _L, _L)] != 0
                gi = plsc.bitcast(g, jnp.int32)
                ordk = jnp.where(gi < 0, gi ^ jnp.int32(0x7FFFFFFF), gi)
                ordk = jnp.where(g == 0.0, 0, ordk)
                key = jnp.where(ab, ordk, jnp.int32(_IMIN))
                keyb[pl.ds(c * _L, _L)] = key
                plsc.addupdate_scatter(hist, [(key >> 20) + 2048], ones)
                nabs_v = nabs_v + plsc.all_reduce_population_count(ab)
                vmax_v = jnp.maximum(vmax_v, key)
            return (nabs_v, vmax_v)
        nabs_v, vmax_v = lax.fori_loop(
            0, _NCH // 4, s1body,
            (jnp.zeros((_L,), jnp.int32), jnp.full((_L,), _IMIN, jnp.int32)))
        nabs = jnp.max(nabs_v)
        kmax = jnp.max(vmax_v)
        k = jnp.minimum(nabs, k_in)

        c0 = ((kmax >> 20) + 2048) // _L

        def s2cond(st):
            c, cum = st
            return (cum < k) & (c >= 0)

        def s2body(st):
            c, cum = st
            return (c - 1, cum + jnp.sum(hist[pl.ds(c * _L, _L)]))

        c_end, cum_end = lax.while_loop(s2cond, s2body, (c0, jnp.int32(0)))
        c_hit = jnp.minimum(c_end + 1, _NBINS // _L - 1)
        h_hit = hist[pl.ds(c_hit * _L, _L)]
        cum_before = cum_end - jnp.sum(h_hit)
        rev = lax.rev(h_hit, (0,))
        hit = (plsc.cumsum(rev) + cum_before) >= k
        lane0_v = plsc.all_reduce_ffs(hit)
        b_star_v = jnp.maximum(c_hit * _L + (_L - 1) - lane0_v, 0)
        bins_hit = c_hit * _L + lanes
        above = cum_before + jnp.sum(jnp.where(bins_hit > b_star_v, h_hit, 0))
        hb_v = plsc.load_gather(hist, [b_star_v])
        need = k - above
        need_v = need
        base_v = (b_star_v - 2048) << 20

        zvec = jnp.zeros((_L,), jnp.int32)
        sep = ((above + jnp.max(hb_v) == k) & (jnp.max(b_star_v) > 0)) \
            | (k == 0)

        def fast_path():
            return base_v - 1, jnp.full((_L,), -1, jnp.int32)

        def slow_path():
            def s3body(cc, off_v):
                for u in range(4):
                    c = cc * 4 + u
                    key = keyb[pl.ds(c * _L, _L)]
                    m = ((key >> 20) + 2048) == b_star_v
                    pos = off_v + plsc.cumsum(m.astype(jnp.int32)) - 1
                    plsc.store_scatter(candk, [pos], key, mask=m)
                    plsc.store_scatter(candi, [pos], lanes + c * _L, mask=m)
                    off_v = off_v + plsc.all_reduce_population_count(m)
                return off_v
            off_v = lax.fori_loop(0, _NCH // 4, s3body, zvec)
            plsc.store_scatter(candk, [off_v + lanes],
                               jnp.full((_L,), _IMIN, jnp.int32))
            nch4 = (jnp.max(off_v) + _L - 1) // _L

            def count_v(pred):
                def cbody(c, acc):
                    kk = candk[pl.ds(c * _L, _L)]
                    ii = candi[pl.ds(c * _L, _L)]
                    return acc + plsc.all_reduce_population_count(pred(kk, ii))
                return lax.fori_loop(0, nch4, cbody, zvec)

            def vdesc(i, prefix_v):
                bit = jnp.int32(1) << (19 - i)
                x_v = base_v | prefix_v | bit
                cnt_v = count_v(lambda kk, ii: kk >= x_v)
                return jnp.where(cnt_v >= need_v, prefix_v | bit, prefix_v)
            prefix_v = lax.fori_loop(0, 20, vdesc, zvec)
            t_v = base_v | prefix_v

            cnt_gt_v = count_v(lambda kk, ii: kk >= t_v + 1)
            ties_v = need_v - cnt_gt_v

            def idesc(i, cut_v):
                bit = jnp.int32(1) << (11 - i)
                ci_v = cut_v + bit - 1
                ce_v = count_v(lambda kk, ii: (kk == t_v) & (ii <= ci_v))
                return jnp.where(ce_v >= ties_v, cut_v, cut_v + bit)
            cut_v = lax.fori_loop(0, 12, idesc, zvec)
            return t_v, cut_v

        t_v, cut_v = lax.cond(sep, fast_path, slow_path)

        def zrow(cc, _):
            for u in range(4):
                hist[pl.ds((cc * 4 + u) * _L, _L)] = zvec
            return 0
        lax.fori_loop(0, _NBINS // _L // 4, zrow, 0)

        tf_v = jnp.where(k > 0, t_v, jnp.int32(_IMAX))
        cf_v = jnp.where(k > 0, cut_v, jnp.int32(-1))
        ov = jnp.where(lanes == 2 * j, tf_v, ov)
        ov = jnp.where(lanes == 2 * j + 1, cf_v, ov)
        return ov

    ov = lax.fori_loop(0, _RPW, row_body, jnp.zeros((_L,), jnp.int32))
    outv[...] = ov
    pltpu.sync_copy(outv.at[pl.ds(0, 8)], out_hbm.at[pl.ds(wid * 8, 8)])


_ROWS = 16
_GRID = _B // _ROWS


def _tc_body(scores_ref, gumbel_ref, absorbed_ref, t_ref, c_ref, out_ref):
    scores = scores_ref[...]
    g = gumbel_ref[...]
    ab = absorbed_ref[...] != 0
    t = t_ref[...]
    cut = c_ref[...]

    gi = lax.bitcast_convert_type(g, jnp.int32)
    ordk = jnp.where(gi < 0, gi ^ jnp.int32(0x7FFFFFFF), gi)
    ordk = jnp.where(g == 0.0, jnp.int32(0), ordk)
    key = jnp.where(ab, ordk, jnp.int32(_IMIN))
    idx = lax.broadcasted_iota(jnp.int32, (_ROWS, _N), 1)
    sel = (key > t) | ((key == t) & (idx <= cut))

    m = jnp.max(scores, axis=1, keepdims=True)
    e = jnp.exp(scores - m)
    s = jnp.sum(e, axis=1, keepdims=True)
    out_ref[...] = jnp.where(sel, e / s, 0.0)


def kernel(scores, gumbel_noise, absorbed_mask, k_per_row):
    kk = k_per_row.astype(jnp.int32)
    sel = _sc_select(gumbel_noise, absorbed_mask, kk).reshape(_B, 2)
    t2 = sel[:, 0:1]
    c2 = sel[:, 1:2]
    return pl.pallas_call(
        _tc_body,
        grid=(_GRID,),
        in_specs=[
            pl.BlockSpec((_ROWS, _N), lambda i: (i, 0)),
            pl.BlockSpec((_ROWS, _N), lambda i: (i, 0)),
            pl.BlockSpec((_ROWS, _N), lambda i: (i, 0)),
            pl.BlockSpec((_ROWS, 1), lambda i: (i, 0)),
            pl.BlockSpec((_ROWS, 1), lambda i: (i, 0)),
        ],
        out_specs=pl.BlockSpec((_ROWS, _N), lambda i: (i, 0)),
        out_shape=jax.ShapeDtypeStruct((_B, _N), jnp.float32),
    )(scores, gumbel_noise, absorbed_mask, t2, c2)
